# Initial kernel scaffold; baseline (speedup 1.0000x reference)
import jax
import jax.numpy as jnp
from jax.experimental import pallas as pl


def kernel(latent):
    raise NotImplementedError("write your pallas kernel here")


# speedup vs baseline: 6.9309x; 6.9309x over previous
import jax
import jax.numpy as jnp
from jax.experimental import pallas as pl

_B = 128
_D = 1024
_G = 8
_CH = 256
_COF1 = 0.01


def _body(row_ref, col_ref, out_ref):
    for s in range(_G):
        row = row_ref[s:s + 1, :]
        acc = jnp.zeros((1, _D), jnp.float32)
        for c in range(_D // _CH):
            col = col_ref[0, pl.ds(c * _CH, _CH), s:s + 1]
            acc = acc + jnp.sum(jnp.abs(col - row), axis=0, keepdims=True)
        t_full = jnp.sum(acc, keepdims=True)
        z = jnp.sum((row == 0.0).astype(jnp.float32), keepdims=True)
        l1 = jnp.sum(jnp.abs(row), keepdims=True)
        out_ref[s:s + 1, :] = t_full - 2.0 * z * l1


def kernel(latent):
    lat_t3 = latent.T.reshape(_D, _B // _G, _G).transpose(1, 0, 2)
    per_sample = pl.pallas_call(
        _body,
        grid=(_B // _G,),
        in_specs=[
            pl.BlockSpec((_G, _D), lambda g: (g, 0)),
            pl.BlockSpec((1, _D, _G), lambda g: (g, 0, 0)),
        ],
        out_specs=pl.BlockSpec((_G, 1), lambda g: (g, 0)),
        out_shape=jax.ShapeDtypeStruct((_B, 1), jnp.float32),
    )(latent, lat_t3)
    return (jnp.sum(per_sample) * (_COF1 / 2.0 / _B)).reshape(())


# speedup vs baseline: 16.7583x; 2.4179x over previous
import functools

import jax
import jax.numpy as jnp
from jax import lax
from jax.experimental import pallas as pl
from jax.experimental.pallas import tpu as pltpu
from jax.experimental.pallas import tpu_sc as plsc

_B = 128
_D = 1024
_L = 16
_NV = _D // _L
_NC = 2
_NS = 16
_NW = _NC * _NS
_SPW = _B // _NW
_COF1 = 0.01


def _asort(t):
    return lax.sort(t)


def _dsort(t):
    return lax.rev(lax.sort(t), (0,))


def _sample_loss(x_v):
    v = [x_v[pl.ds(i * _L, _L)] for i in range(_NV)]

    zc = jnp.zeros((_L,), jnp.float32)
    l1v = jnp.zeros((_L,), jnp.float32)
    for i in range(_NV):
        l1v = l1v + jnp.abs(v[i])
        zc = zc + (1.0 - jnp.abs(jnp.sign(v[i])))
    z_vec = plsc.cumsum(zc)
    l1_vec = plsc.cumsum(l1v)

    for i in range(_NV):
        v[i] = _asort(v[i]) if i % 2 == 0 else _dsort(v[i])

    for m in range(1, 7):
        run = 1 << m
        for r in range(_NV // run):
            asc = True if m == 6 else (r % 2 == 0)
            base = r * run
            dist = run // 2
            while dist >= 1:
                for g0 in range(base, base + run, 2 * dist):
                    for a in range(g0, g0 + dist):
                        b = a + dist
                        mn = jnp.minimum(v[a], v[b])
                        mx = jnp.maximum(v[a], v[b])
                        v[a], v[b] = (mn, mx) if asc else (mx, mn)
                dist //= 2
            for i in range(base, base + run):
                v[i] = _asort(v[i]) if asc else _dsort(v[i])

    iota2 = 2.0 * lax.iota(jnp.int32, _L).astype(jnp.float32)
    acc = jnp.zeros((_L,), jnp.float32)
    for i in range(_NV):
        acc = acc + v[i] * (iota2 + float(2 * _L * i + 1 - _D))
    t_vec = 2.0 * plsc.cumsum(acc)
    return t_vec - 2.0 * z_vec * l1_vec


@functools.partial(
    pl.kernel,
    mesh=plsc.VectorSubcoreMesh(core_axis_name="c", subcore_axis_name="s"),
    out_type=jax.ShapeDtypeStruct((_NW, _L), jnp.float32),
    scratch_types=[
        pltpu.VMEM((_D,), jnp.float32),
        pltpu.VMEM((_L,), jnp.float32),
    ],
    compiler_params=pltpu.CompilerParams(needs_layout_passes=False),
)
def _sc_body(lat_hbm, out_hbm, x_v, o_v):
    wid = lax.axis_index("s") * _NC + lax.axis_index("c")

    def body(i, tot_vec):
        pltpu.sync_copy(lat_hbm.at[wid * _SPW + i], x_v)
        return tot_vec + _sample_loss(x_v)

    tot_vec = lax.fori_loop(0, _SPW, body, jnp.zeros((_L,), jnp.float32))
    o_v[...] = tot_vec
    pltpu.sync_copy(o_v, out_hbm.at[wid])


def kernel(latent):
    partials = _sc_body(latent)
    return (jnp.sum(partials[:, _L - 1]) * (_COF1 / 2.0 / _B)).reshape(())


# speedup vs baseline: 27.7349x; 1.6550x over previous
import jax
import jax.numpy as jnp
from jax.experimental import pallas as pl

_B = 128
_D = 1024
_COF1 = 0.01


def _tc_body(xt_ref, out_ref):
    x = xt_ref[...]
    row = jax.lax.broadcasted_iota(jnp.int32, (_D, 1), 0)

    zl = jnp.sum(1.0 - jnp.abs(jnp.sign(x)), axis=0, keepdims=True)
    l1 = jnp.sum(jnp.abs(x), axis=0, keepdims=True)

    for s in range(10):
        asc_col = ((row >> (s + 1)) & 1) == 0
        for p in range(s, -1, -1):
            j = 1 << p
            up = jnp.concatenate([x[j:, :], x[:j, :]], axis=0)
            down = jnp.concatenate([x[_D - j:, :], x[:_D - j, :]], axis=0)
            low_col = (row & j) == 0
            partner = jnp.where(low_col, up, down)
            mn = jnp.minimum(x, partner)
            mx = jnp.maximum(x, partner)
            x = jnp.where(low_col == asc_col, mn, mx)

    w = (2.0 * row.astype(jnp.float32) + float(1 - _D))
    t_full = 2.0 * jnp.sum(x * w, axis=0, keepdims=True)
    out_ref[...] = t_full - 2.0 * zl * l1


def kernel(latent):
    per_sample = pl.pallas_call(
        _tc_body,
        out_shape=jax.ShapeDtypeStruct((1, _B), jnp.float32),
    )(latent.T)
    return (jnp.sum(per_sample) * (_COF1 / 2.0 / _B)).reshape(())


# speedup vs baseline: 37.1241x; 1.3385x over previous
import jax
import jax.numpy as jnp
from jax.experimental import pallas as pl

_B = 128
_D = 1024
_COF1 = 0.01


def _tc_body(lat_ref, out_ref):
    x = lat_ref[...].T
    row = jax.lax.broadcasted_iota(jnp.int32, (_D, 1), 0)

    zl = jnp.sum(jnp.where(x == 0.0, 1.0, 0.0), axis=0, keepdims=True)
    l1 = jnp.sum(jnp.abs(x), axis=0, keepdims=True)

    for s in range(10):
        asc_col = ((row >> (s + 1)) & 1) == 0
        for p in range(s, -1, -1):
            j = 1 << p
            up = jnp.concatenate([x[j:, :], x[:j, :]], axis=0)
            down = jnp.concatenate([x[_D - j:, :], x[:_D - j, :]], axis=0)
            low_col = (row & j) == 0
            partner = jnp.where(low_col, up, down)
            mn = jnp.minimum(x, partner)
            mx = jnp.maximum(x, partner)
            x = jnp.where(low_col == asc_col, mn, mx)

    w = (2.0 * row.astype(jnp.float32) + float(1 - _D))
    t_full = 2.0 * jnp.sum(x * w, axis=0, keepdims=True)
    m_vec = t_full - 2.0 * zl * l1
    out_ref[...] = jnp.sum(m_vec, axis=1, keepdims=True) * (_COF1 / 2.0 / _B)


def kernel(latent):
    loss = pl.pallas_call(
        _tc_body,
        out_shape=jax.ShapeDtypeStruct((1, 1), jnp.float32),
    )(latent)
    return loss.reshape(())


# speedup vs baseline: 50.6163x; 1.3634x over previous
import jax
import jax.numpy as jnp
from jax.experimental import pallas as pl
from jax.experimental.pallas import tpu as pltpu

_B = 128
_D = 1024
_COF1 = 0.01


def _tc_body(lat_ref, out_ref, xs_ref):
    xs_ref[...] = lat_ref[...].T
    row = jax.lax.broadcasted_iota(jnp.int32, (_D, 1), 0)
    low_masks = [(row & (1 << p)) == 0 for p in range(3)]

    x0 = xs_ref[...]
    zl = jnp.sum(jnp.where(x0 == 0.0, 1.0, 0.0), axis=0, keepdims=True)
    l1 = jnp.sum(jnp.abs(x0), axis=0, keepdims=True)

    for s in range(10):
        for p in range(s, -1, -1):
            j = 1 << p
            if p >= 3:
                for a in range(0, _D, 2 * j):
                    asc = ((a >> (s + 1)) & 1) == 0
                    lo = xs_ref[pl.ds(a, j), :]
                    hi = xs_ref[pl.ds(a + j, j), :]
                    mn = jnp.minimum(lo, hi)
                    mx = jnp.maximum(lo, hi)
                    xs_ref[pl.ds(a, j), :] = mn if asc else mx
                    xs_ref[pl.ds(a + j, j), :] = mx if asc else mn
            else:
                x = xs_ref[...]
                up = jnp.concatenate([x[j:, :], x[:j, :]], axis=0)
                down = jnp.concatenate([x[_D - j:, :], x[:_D - j, :]], axis=0)
                low = low_masks[p]
                partner = jnp.where(low, up, down)
                mn = jnp.minimum(x, partner)
                mx = jnp.maximum(x, partner)
                if s + 1 >= 3:
                    run = 1 << (s + 1)
                    pieces = []
                    for a in range(0, _D, run):
                        asc = ((a >> (s + 1)) & 1) == 0
                        lw = low[a:a + run, :]
                        mnr = mn[a:a + run, :]
                        mxr = mx[a:a + run, :]
                        pieces.append(jnp.where(lw, mnr, mxr) if asc
                                      else jnp.where(lw, mxr, mnr))
                    xs_ref[...] = jnp.concatenate(pieces, axis=0)
                else:
                    sel = (((row >> p) ^ (row >> (s + 1))) & 1) == 0
                    xs_ref[...] = jnp.where(sel, mn, mx)

    xs = xs_ref[...]
    w = 2.0 * row.astype(jnp.float32) + float(1 - _D)
    t_full = 2.0 * jnp.sum(xs * w, axis=0, keepdims=True)
    m_vec = t_full - 2.0 * zl * l1
    out_ref[...] = jnp.sum(m_vec, axis=1, keepdims=True) * (_COF1 / 2.0 / _B)


def kernel(latent):
    loss = pl.pallas_call(
        _tc_body,
        out_shape=jax.ShapeDtypeStruct((1, 1), jnp.float32),
        scratch_shapes=[pltpu.VMEM((_D, _B), jnp.float32)],
    )(latent)
    return loss.reshape(())


# speedup vs baseline: 59.0686x; 1.1670x over previous
import jax
import jax.numpy as jnp
from jax.experimental import pallas as pl
from jax.experimental.pallas import tpu as pltpu

_B = 128
_D = 1024
_COF1 = 0.01


def _rolls(x, j):
    up = jnp.concatenate([x[j:, :], x[:j, :]], axis=0)
    down = jnp.concatenate([x[_D - j:, :], x[:_D - j, :]], axis=0)
    return up, down


def _small_chain(x, steps, low_masks, row):
    for s, p in steps:
        j = 1 << p
        up, down = _rolls(x, j)
        if s + 1 >= 3:
            run = 1 << (s + 1)
            pieces = []
            for a in range(0, _D, run):
                asc = ((a >> (s + 1)) & 1) == 0
                lw = low_masks[p][a:a + run, :]
                xr = x[a:a + run, :]
                upr = up[a:a + run, :]
                dnr = down[a:a + run, :]
                if asc:
                    pieces.append(jnp.where(lw, jnp.minimum(xr, upr),
                                            jnp.maximum(xr, dnr)))
                else:
                    pieces.append(jnp.where(lw, jnp.maximum(xr, upr),
                                            jnp.minimum(xr, dnr)))
            x = jnp.concatenate(pieces, axis=0)
        else:
            sel = (((row >> p) ^ (row >> (s + 1))) & 1) == 0
            partner = jnp.where(low_masks[p], up, down)
            x = jnp.where(sel, jnp.minimum(x, partner), jnp.maximum(x, partner))
    return x


def _block_exchange(blk, rows, j, asc):
    pieces = []
    for b in range(0, rows, 2 * j):
        lo = blk[b:b + j, :]
        hi = blk[b + j:b + 2 * j, :]
        mn = jnp.minimum(lo, hi)
        mx = jnp.maximum(lo, hi)
        pieces.extend([mn, mx] if asc else [mx, mn])
    return jnp.concatenate(pieces, axis=0)


def _tc_body(lat_ref, out_ref, xs_ref):
    row = jax.lax.broadcasted_iota(jnp.int32, (_D, 1), 0)
    low_masks = [(row & (1 << p)) == 0 for p in range(3)]

    x = lat_ref[...].T
    zl = jnp.sum(jnp.where(x == 0.0, 1.0, 0.0), axis=0, keepdims=True)
    l1 = jnp.sum(jnp.abs(x), axis=0, keepdims=True)

    x = _small_chain(x, ((0, 0), (1, 1), (1, 0), (2, 2), (2, 1), (2, 0)),
                     low_masks, row)
    xs_ref[...] = x

    for s in range(3, 10):
        ptop = min(s, 5)
        for p in range(s, ptop, -1):
            j = 1 << p
            for a in range(0, _D, 2 * j):
                asc = ((a >> (s + 1)) & 1) == 0
                lo = xs_ref[pl.ds(a, j), :]
                hi = xs_ref[pl.ds(a + j, j), :]
                mn = jnp.minimum(lo, hi)
                mx = jnp.maximum(lo, hi)
                xs_ref[pl.ds(a, j), :] = mn if asc else mx
                xs_ref[pl.ds(a + j, j), :] = mx if asc else mn
        sb = 2 << ptop
        for a in range(0, _D, sb):
            asc = ((a >> (s + 1)) & 1) == 0
            blk = xs_ref[pl.ds(a, sb), :]
            for p in range(ptop, 2, -1):
                blk = _block_exchange(blk, sb, 1 << p, asc)
            xs_ref[pl.ds(a, sb), :] = blk
        x = xs_ref[...]
        x = _small_chain(x, ((s, 2), (s, 1), (s, 0)), low_masks, row)
        if s < 9:
            xs_ref[...] = x

    w = 2.0 * row.astype(jnp.float32) + float(1 - _D)
    t_full = 2.0 * jnp.sum(x * w, axis=0, keepdims=True)
    m_vec = t_full - 2.0 * zl * l1
    out_ref[...] = jnp.sum(m_vec, axis=1, keepdims=True) * (_COF1 / 2.0 / _B)


def kernel(latent):
    loss = pl.pallas_call(
        _tc_body,
        out_shape=jax.ShapeDtypeStruct((1, 1), jnp.float32),
        scratch_shapes=[pltpu.VMEM((_D, _B), jnp.float32)],
    )(latent)
    return loss.reshape(())


# speedup vs baseline: 65.9710x; 1.1169x over previous
import jax
import jax.numpy as jnp
from jax.experimental import pallas as pl
from jax.experimental.pallas import tpu as pltpu

_B = 128
_D = 1024
_SB = 64
_COF1 = 0.01


def _roll_pair(x, j, rows):
    up = jnp.concatenate([x[j:, :], x[:j, :]], axis=0)
    down = jnp.concatenate([x[rows - j:, :], x[:rows - j, :]], axis=0)
    return up, down


def _block_exchange(blk, rows, j, asc):
    pieces = []
    for b in range(0, rows, 2 * j):
        lo = blk[b:b + j, :]
        hi = blk[b + j:b + 2 * j, :]
        mn = jnp.minimum(lo, hi)
        mx = jnp.maximum(lo, hi)
        pieces.extend([mn, mx] if asc else [mx, mn])
    return jnp.concatenate(pieces, axis=0)


def _small_exchange(blk, rows, j, asc, low):
    up, down = _roll_pair(blk, j, rows)
    if asc:
        return jnp.where(low, jnp.minimum(blk, up), jnp.maximum(blk, down))
    return jnp.where(low, jnp.maximum(blk, up), jnp.minimum(blk, down))


def _tc_body(lat_ref, out_ref, xs_ref):
    row = jax.lax.broadcasted_iota(jnp.int32, (_D, 1), 0)
    low_masks = [(row & (1 << p)) == 0 for p in range(3)]
    low_sb = [m[:_SB, :] for m in low_masks]

    x = lat_ref[...].T
    zl = jnp.sum(jnp.where(x == 0.0, 1.0, 0.0), axis=0, keepdims=True)
    l1 = jnp.sum(jnp.abs(x), axis=0, keepdims=True)

    for s, p in ((0, 0), (1, 1), (1, 0), (2, 2), (2, 1), (2, 0)):
        j = 1 << p
        up, down = _roll_pair(x, j, _D)
        sel = (((row >> p) ^ (row >> (s + 1))) & 1) == 0
        partner = jnp.where(low_masks[p], up, down)
        x = jnp.where(sel, jnp.minimum(x, partner), jnp.maximum(x, partner))
    xs_ref[...] = x

    w = 2.0 * row.astype(jnp.float32) + float(1 - _D)
    t_acc = jnp.zeros((1, _B), jnp.float32)

    for s in range(3, 10):
        sb = min(_SB, 1 << (s + 1))
        ptop = min(s, 5)
        for p in range(s, ptop, -1):
            j = 1 << p
            for a in range(0, _D, 2 * j):
                asc = ((a >> (s + 1)) & 1) == 0
                lo = xs_ref[pl.ds(a, j), :]
                hi = xs_ref[pl.ds(a + j, j), :]
                mn = jnp.minimum(lo, hi)
                mx = jnp.maximum(lo, hi)
                xs_ref[pl.ds(a, j), :] = mn if asc else mx
                xs_ref[pl.ds(a + j, j), :] = mx if asc else mn
        for a in range(0, _D, sb):
            asc = ((a >> (s + 1)) & 1) == 0
            blk = xs_ref[pl.ds(a, sb), :]
            for p in range(ptop, 2, -1):
                blk = _block_exchange(blk, sb, 1 << p, asc)
            for p in (2, 1, 0):
                blk = _small_exchange(blk, sb, 1 << p, asc, low_sb[p][:sb, :])
            if s < 9:
                xs_ref[pl.ds(a, sb), :] = blk
            else:
                t_acc = t_acc + jnp.sum(blk * w[a:a + sb, :], axis=0,
                                        keepdims=True)

    m_vec = 2.0 * t_acc - 2.0 * zl * l1
    out_ref[...] = jnp.sum(m_vec, axis=1, keepdims=True) * (_COF1 / 2.0 / _B)


def kernel(latent):
    loss = pl.pallas_call(
        _tc_body,
        out_shape=jax.ShapeDtypeStruct((1, 1), jnp.float32),
        scratch_shapes=[pltpu.VMEM((_D, _B), jnp.float32)],
    )(latent)
    return loss.reshape(())


# speedup vs baseline: 82.3664x; 1.2485x over previous
import jax
import jax.numpy as jnp
from jax.experimental import pallas as pl
from jax.experimental.pallas import tpu as pltpu

_B = 128
_D = 1024
_SB = 64
_COF1 = 0.01


def _grot(x, rows, j):
    x3 = x.reshape(rows // 8, 8, _B)
    jj = j % 8
    r = jnp.concatenate([x3[:, jj:, :], x3[:, :jj, :]], axis=1)
    return r.reshape(rows, _B)


def _xor_partner(x, rows, j, low):
    if j == 4:
        return _grot(x, rows, 4)
    return jnp.where(low, _grot(x, rows, j), _grot(x, rows, -j))


def _block_exchange(blk, rows, j, asc):
    pieces = []
    for b in range(0, rows, 2 * j):
        lo = blk[b:b + j, :]
        hi = blk[b + j:b + 2 * j, :]
        mn = jnp.minimum(lo, hi)
        mx = jnp.maximum(lo, hi)
        pieces.extend([mn, mx] if asc else [mx, mn])
    return jnp.concatenate(pieces, axis=0)


def _small_exchange(blk, rows, j, asc, low):
    if j == 4:
        partner = _grot(blk, rows, 4)
    else:
        up = _grot(blk, rows, j)
        down = _grot(blk, rows, -j)
        if asc:
            return jnp.where(low, jnp.minimum(blk, up), jnp.maximum(blk, down))
        return jnp.where(low, jnp.maximum(blk, up), jnp.minimum(blk, down))
    mn = jnp.minimum(blk, partner)
    mx = jnp.maximum(blk, partner)
    return jnp.where(low, mn, mx) if asc else jnp.where(low, mx, mn)


def _tc_body(lat_ref, out_ref, xs_ref):
    row = jax.lax.broadcasted_iota(jnp.int32, (_D, 1), 0)
    low_masks = [(row & (1 << p)) == 0 for p in range(3)]
    low_sb = [m[:_SB, :] for m in low_masks]

    x = lat_ref[...].T
    zl = jnp.sum(jnp.where(x == 0.0, 1.0, 0.0), axis=0, keepdims=True)
    l1 = jnp.sum(jnp.abs(x), axis=0, keepdims=True)

    for s, p in ((0, 0), (1, 1), (1, 0), (2, 2), (2, 1), (2, 0)):
        j = 1 << p
        sel = (((row >> p) ^ (row >> (s + 1))) & 1) == 0
        partner = _xor_partner(x, _D, j, low_masks[p])
        x = jnp.where(sel, jnp.minimum(x, partner), jnp.maximum(x, partner))
    xs_ref[...] = x

    w = 2.0 * row.astype(jnp.float32) + float(1 - _D)
    t_acc = jnp.zeros((1, _B), jnp.float32)

    for s in range(3, 10):
        sb = min(_SB, 1 << (s + 1))
        ptop = min(s, 5)
        for p in range(s, ptop, -1):
            j = 1 << p
            for a in range(0, _D, 2 * j):
                asc = ((a >> (s + 1)) & 1) == 0
                lo = xs_ref[pl.ds(a, j), :]
                hi = xs_ref[pl.ds(a + j, j), :]
                mn = jnp.minimum(lo, hi)
                mx = jnp.maximum(lo, hi)
                xs_ref[pl.ds(a, j), :] = mn if asc else mx
                xs_ref[pl.ds(a + j, j), :] = mx if asc else mn
        for a in range(0, _D, sb):
            asc = ((a >> (s + 1)) & 1) == 0
            blk = xs_ref[pl.ds(a, sb), :]
            for p in range(ptop, 2, -1):
                blk = _block_exchange(blk, sb, 1 << p, asc)
            for p in (2, 1, 0):
                blk = _small_exchange(blk, sb, 1 << p, asc, low_sb[p][:sb, :])
            if s < 9:
                xs_ref[pl.ds(a, sb), :] = blk
            else:
                t_acc = t_acc + jnp.sum(blk * w[a:a + sb, :], axis=0,
                                        keepdims=True)

    m_vec = 2.0 * t_acc - 2.0 * zl * l1
    out_ref[...] = jnp.sum(m_vec, axis=1, keepdims=True) * (_COF1 / 2.0 / _B)


def kernel(latent):
    loss = pl.pallas_call(
        _tc_body,
        out_shape=jax.ShapeDtypeStruct((1, 1), jnp.float32),
        scratch_shapes=[pltpu.VMEM((_D, _B), jnp.float32)],
    )(latent)
    return loss.reshape(())


# speedup vs baseline: 89.0534x; 1.0812x over previous
import jax
import jax.numpy as jnp
from jax.experimental import pallas as pl
from jax.experimental.pallas import tpu as pltpu

_B = 128
_D = 1024
_SB = 64
_COF1 = 0.01


def _phi(b):
    return b + 3 if b <= 6 else b - 7


def _grot(x, rows, j):
    x3 = x.reshape(rows // 8, 8, _B)
    jj = j % 8
    r = jnp.concatenate([x3[:, jj:, :], x3[:, :jj, :]], axis=1)
    return r.reshape(rows, _B)


def _xor_partner(x, rows, j, low):
    if j == 4:
        return _grot(x, rows, 4)
    return jnp.where(low, _grot(x, rows, j), _grot(x, rows, -j))


def _blk_step(blk, rows, a0, pd, dird, ascm):
    j = 1 << pd
    pieces = []
    for b in range(0, rows, 2 * j):
        lo = blk[b:b + j, :]
        hi = blk[b + j:b + 2 * j, :]
        mn = jnp.minimum(lo, hi)
        mx = jnp.maximum(lo, hi)
        if dird is None:
            pieces.extend([mn, mx])
        elif dird >= 3:
            asc = (((a0 + b) >> dird) & 1) == 0
            pieces.extend([mn, mx] if asc else [mx, mn])
        else:
            am = jnp.concatenate([ascm[dird]] * (j // 8), axis=0)
            pieces.append(jnp.where(am, mn, mx))
            pieces.append(jnp.where(am, mx, mn))
    return jnp.concatenate(pieces, axis=0)


def _tc_body(lat_ref, out_ref, xs_ref):
    row = jax.lax.broadcasted_iota(jnp.int32, (_D, 1), 0)
    low_masks = [(row & (1 << pd)) == 0 for pd in range(3)]
    ascm = [((row[:8, :] >> d) & 1) == 0 for d in range(3)]

    x = lat_ref[...].T
    zl = jnp.sum(jnp.where(x == 0.0, 1.0, 0.0), axis=0, keepdims=True)
    l1 = jnp.sum(jnp.abs(x), axis=0, keepdims=True)

    sigma = ((row & 7) << 7) | (row >> 3)
    w = 2.0 * sigma.astype(jnp.float32) + float(1 - _D)

    for a in range(0, _D, _SB):
        blk = x[a:a + _SB, :]
        for s, p in ((0, 0), (1, 1), (1, 0), (2, 2), (2, 1), (2, 0)):
            blk = _blk_step(blk, _SB, a, _phi(p), _phi(s + 1), ascm)
        xs_ref[pl.ds(a, _SB), :] = blk

    t_acc = jnp.zeros((1, _B), jnp.float32)

    for s in range(3, 10):
        dird = _phi(s + 1) if s < 9 else None
        for p in range(s, 6, -1):
            pd = _phi(p)
            xv = xs_ref[...]
            partner = _xor_partner(xv, _D, 1 << pd, low_masks[pd])
            mn = jnp.minimum(xv, partner)
            mx = jnp.maximum(xv, partner)
            if dird is None:
                sel = low_masks[pd]
            else:
                sel = low_masks[pd] == (((row >> dird) & 1) == 0)
            xs_ref[...] = jnp.where(sel, mn, mx)
        for p in range(min(s, 6), 2, -1):
            j = 1 << _phi(p)
            for a in range(0, _D, 2 * j):
                lo = xs_ref[pl.ds(a, j), :]
                hi = xs_ref[pl.ds(a + j, j), :]
                mn = jnp.minimum(lo, hi)
                mx = jnp.maximum(lo, hi)
                if dird is None or dird >= 3:
                    asc = dird is None or ((a >> dird) & 1) == 0
                    xs_ref[pl.ds(a, j), :] = mn if asc else mx
                    xs_ref[pl.ds(a + j, j), :] = mx if asc else mn
                else:
                    amj = jnp.concatenate([ascm[dird]] * (j // 8), axis=0)
                    xs_ref[pl.ds(a, j), :] = jnp.where(amj, mn, mx)
                    xs_ref[pl.ds(a + j, j), :] = jnp.where(amj, mx, mn)
        for a in range(0, _D, _SB):
            blk = xs_ref[pl.ds(a, _SB), :]
            for p in (2, 1, 0):
                blk = _blk_step(blk, _SB, a, _phi(p), dird, ascm)
            if s < 9:
                xs_ref[pl.ds(a, _SB), :] = blk
            else:
                t_acc = t_acc + jnp.sum(blk * w[a:a + _SB, :], axis=0,
                                        keepdims=True)

    m_vec = 2.0 * t_acc - 2.0 * zl * l1
    out_ref[...] = jnp.sum(m_vec, axis=1, keepdims=True) * (_COF1 / 2.0 / _B)


def kernel(latent):
    loss = pl.pallas_call(
        _tc_body,
        out_shape=jax.ShapeDtypeStruct((1, 1), jnp.float32),
        scratch_shapes=[pltpu.VMEM((_D, _B), jnp.float32)],
    )(latent)
    return loss.reshape(())


# speedup vs baseline: 89.6999x; 1.0073x over previous
import jax
import jax.numpy as jnp
from jax.experimental import pallas as pl
from jax.experimental.pallas import tpu as pltpu

_B = 128
_D = 1024
_SB = 128
_COF1 = 0.01


def _phi(b):
    return b + 3 if b <= 6 else b - 7


def _grot(x, rows, j):
    x3 = x.reshape(rows // 8, 8, _B)
    jj = j % 8
    r = jnp.concatenate([x3[:, jj:, :], x3[:, :jj, :]], axis=1)
    return r.reshape(rows, _B)


def _xor_partner(x, rows, j, low):
    if j == 4:
        return _grot(x, rows, 4)
    return jnp.where(low, _grot(x, rows, j), _grot(x, rows, -j))


def _blk_step(blk, rows, a0, pd, dird, ascm):
    j = 1 << pd
    pieces = []
    for b in range(0, rows, 2 * j):
        lo = blk[b:b + j, :]
        hi = blk[b + j:b + 2 * j, :]
        mn = jnp.minimum(lo, hi)
        mx = jnp.maximum(lo, hi)
        if dird is None:
            pieces.extend([mn, mx])
        elif dird >= 3:
            asc = (((a0 + b) >> dird) & 1) == 0
            pieces.extend([mn, mx] if asc else [mx, mn])
        else:
            am = jnp.concatenate([ascm[dird]] * (j // 8), axis=0)
            pieces.append(jnp.where(am, mn, mx))
            pieces.append(jnp.where(am, mx, mn))
    return jnp.concatenate(pieces, axis=0)


def _tc_body(lat_ref, out_ref, xs_ref):
    row = jax.lax.broadcasted_iota(jnp.int32, (_D, 1), 0)
    low_masks = [(row & (1 << pd)) == 0 for pd in range(3)]
    ascm = [((row[:8, :] >> d) & 1) == 0 for d in range(3)]

    x = lat_ref[...].T
    zl = jnp.sum(jnp.where(x == 0.0, 1.0, 0.0), axis=0, keepdims=True)
    l1 = jnp.sum(jnp.abs(x), axis=0, keepdims=True)

    sigma = ((row & 7) << 7) | (row >> 3)
    w = 2.0 * sigma.astype(jnp.float32) + float(1 - _D)

    for a in range(0, _D, _SB):
        blk = x[a:a + _SB, :]
        for s, p in ((0, 0), (1, 1), (1, 0), (2, 2), (2, 1), (2, 0)):
            blk = _blk_step(blk, _SB, a, _phi(p), _phi(s + 1), ascm)
        xs_ref[pl.ds(a, _SB), :] = blk

    t_acc = jnp.zeros((1, _B), jnp.float32)

    for s in range(3, 10):
        dird = _phi(s + 1) if s < 9 else None
        if s >= 7:
            xv = xs_ref[...]
            for p in range(s, 6, -1):
                pd = _phi(p)
                partner = _xor_partner(xv, _D, 1 << pd, low_masks[pd])
                mn = jnp.minimum(xv, partner)
                mx = jnp.maximum(xv, partner)
                if dird is None:
                    sel = low_masks[pd]
                else:
                    sel = low_masks[pd] == (((row >> dird) & 1) == 0)
                xv = jnp.where(sel, mn, mx)
            xs_ref[...] = xv
        for p in range(min(s, 6), 3, -1):
            j = 1 << _phi(p)
            for a in range(0, _D, 2 * j):
                lo = xs_ref[pl.ds(a, j), :]
                hi = xs_ref[pl.ds(a + j, j), :]
                mn = jnp.minimum(lo, hi)
                mx = jnp.maximum(lo, hi)
                if dird is None or dird >= 3:
                    asc = dird is None or ((a >> dird) & 1) == 0
                    xs_ref[pl.ds(a, j), :] = mn if asc else mx
                    xs_ref[pl.ds(a + j, j), :] = mx if asc else mn
                else:
                    amj = jnp.concatenate([ascm[dird]] * (j // 8), axis=0)
                    xs_ref[pl.ds(a, j), :] = jnp.where(amj, mn, mx)
                    xs_ref[pl.ds(a + j, j), :] = jnp.where(amj, mx, mn)
        for a in range(0, _D, _SB):
            blk = xs_ref[pl.ds(a, _SB), :]
            for p in ((3, 2, 1, 0) if s >= 3 else (2, 1, 0)):
                blk = _blk_step(blk, _SB, a, _phi(p), dird, ascm)
            if s < 9:
                xs_ref[pl.ds(a, _SB), :] = blk
            else:
                t_acc = t_acc + jnp.sum(blk * w[a:a + _SB, :], axis=0,
                                        keepdims=True)

    m_vec = 2.0 * t_acc - 2.0 * zl * l1
    out_ref[...] = jnp.sum(m_vec, axis=1, keepdims=True) * (_COF1 / 2.0 / _B)


def kernel(latent):
    loss = pl.pallas_call(
        _tc_body,
        out_shape=jax.ShapeDtypeStruct((1, 1), jnp.float32),
        scratch_shapes=[pltpu.VMEM((_D, _B), jnp.float32)],
    )(latent)
    return loss.reshape(())


# speedup vs baseline: 101.4686x; 1.1312x over previous
import jax
import jax.numpy as jnp
from jax.experimental import pallas as pl
from jax.experimental.pallas import tpu as pltpu

_B = 128
_D = 1024
_SB = 128
_COF1 = 0.01


def _phi(b):
    return b + 3 if b <= 6 else b - 7


def _grot(x, rows, j):
    x3 = x.reshape(rows // 8, 8, _B)
    jj = j % 8
    r = jnp.concatenate([x3[:, jj:, :], x3[:, :jj, :]], axis=1)
    return r.reshape(rows, _B)


def _xor_partner(x, rows, j, low):
    if j == 4:
        return _grot(x, rows, 4)
    return jnp.where(low, _grot(x, rows, j), _grot(x, rows, -j))


def _blk_step(blk, rows, a0, pd, dird, ascm):
    j = 1 << pd
    pieces = []
    for b in range(0, rows, 2 * j):
        lo = blk[b:b + j, :]
        hi = blk[b + j:b + 2 * j, :]
        mn = jnp.minimum(lo, hi)
        mx = jnp.maximum(lo, hi)
        if dird is None:
            pieces.extend([mn, mx])
        elif dird >= 3:
            asc = (((a0 + b) >> dird) & 1) == 0
            pieces.extend([mn, mx] if asc else [mx, mn])
        else:
            am = jnp.concatenate([ascm[dird]] * (j // 8), axis=0)
            pieces.append(jnp.where(am, mn, mx))
            pieces.append(jnp.where(am, mx, mn))
    return jnp.concatenate(pieces, axis=0)


def _tc_body(lat_ref, out_ref, xs_ref):
    row = jax.lax.broadcasted_iota(jnp.int32, (_D, 1), 0)
    low_masks = [(row & (1 << pd)) == 0 for pd in range(3)]
    ascm = [((row[:8, :] >> d) & 1) == 0 for d in range(3)]

    x = lat_ref[...].T
    zl = jnp.sum(jnp.where(x == 0.0, 1.0, 0.0), axis=0, keepdims=True)
    l1 = jnp.sum(jnp.abs(x), axis=0, keepdims=True)

    sigma = ((row & 7) << 7) | (row >> 3)
    w = 2.0 * sigma.astype(jnp.float32) + float(1 - _D)

    for a in range(0, _D, _SB):
        blk = x[a:a + _SB, :]
        for s, p in ((0, 0), (1, 1), (1, 0), (2, 2), (2, 1), (2, 0),
                     (3, 3), (3, 2), (3, 1), (3, 0)):
            blk = _blk_step(blk, _SB, a, _phi(p), _phi(s + 1), ascm)
        xs_ref[pl.ds(a, _SB), :] = blk

    t_acc = jnp.zeros((1, _B), jnp.float32)

    def grot_chain(xv, rows, s, dird):
        for p in range(s, 6, -1):
            pd = _phi(p)
            partner = _xor_partner(xv, rows, 1 << pd, low_masks[pd][:rows, :])
            mn = jnp.minimum(xv, partner)
            mx = jnp.maximum(xv, partner)
            if dird is None:
                sel = low_masks[pd][:rows, :]
            else:
                sel = (low_masks[pd] == (((row >> dird) & 1) == 0))[:rows, :]
            xv = jnp.where(sel, mn, mx)
        return xv

    for s in range(4, 10):
        dird = _phi(s + 1) if s < 9 else None
        if s >= 7:
            lo = grot_chain(xs_ref[pl.ds(0, 512), :], 512, s, dird)
            hi = grot_chain(xs_ref[pl.ds(512, 512), :], 512, s, dird)
            mn = jnp.minimum(lo, hi)
            mx = jnp.maximum(lo, hi)
            if dird is None:
                xs_ref[pl.ds(0, 512), :] = mn
                xs_ref[pl.ds(512, 512), :] = mx
            else:
                am = (((row >> dird) & 1) == 0)[:512, :]
                xs_ref[pl.ds(0, 512), :] = jnp.where(am, mn, mx)
                xs_ref[pl.ds(512, 512), :] = jnp.where(am, mx, mn)
        for p in range(min(s, 6) if s < 7 else 5, 3, -1):
            j = 1 << _phi(p)
            for a in range(0, _D, 2 * j):
                lo = xs_ref[pl.ds(a, j), :]
                hi = xs_ref[pl.ds(a + j, j), :]
                mn = jnp.minimum(lo, hi)
                mx = jnp.maximum(lo, hi)
                if dird is None or dird >= 3:
                    asc = dird is None or ((a >> dird) & 1) == 0
                    xs_ref[pl.ds(a, j), :] = mn if asc else mx
                    xs_ref[pl.ds(a + j, j), :] = mx if asc else mn
                else:
                    amj = jnp.concatenate([ascm[dird]] * (j // 8), axis=0)
                    xs_ref[pl.ds(a, j), :] = jnp.where(amj, mn, mx)
                    xs_ref[pl.ds(a + j, j), :] = jnp.where(amj, mx, mn)
        for a in range(0, _D, _SB):
            blk = xs_ref[pl.ds(a, _SB), :]
            for p in ((3, 2, 1, 0) if s >= 3 else (2, 1, 0)):
                blk = _blk_step(blk, _SB, a, _phi(p), dird, ascm)
            if s < 9:
                xs_ref[pl.ds(a, _SB), :] = blk
            else:
                t_acc = t_acc + jnp.sum(blk * w[a:a + _SB, :], axis=0,
                                        keepdims=True)

    m_vec = 2.0 * t_acc - 2.0 * zl * l1
    out_ref[...] = jnp.sum(m_vec, axis=1, keepdims=True) * (_COF1 / 2.0 / _B)


def kernel(latent):
    loss = pl.pallas_call(
        _tc_body,
        out_shape=jax.ShapeDtypeStruct((1, 1), jnp.float32),
        scratch_shapes=[pltpu.VMEM((_D, _B), jnp.float32)],
    )(latent)
    return loss.reshape(())


# speedup vs baseline: 102.1354x; 1.0066x over previous
import jax
import jax.numpy as jnp
from jax.experimental import pallas as pl
from jax.experimental.pallas import tpu as pltpu

_B = 128
_D = 1024
_SB = 128
_COF1 = 0.01


def _phi(b):
    return b + 3 if b <= 6 else b - 7


def _grot(x, rows, j):
    x3 = x.reshape(rows // 8, 8, _B)
    jj = j % 8
    r = jnp.concatenate([x3[:, jj:, :], x3[:, :jj, :]], axis=1)
    return r.reshape(rows, _B)


def _xor_partner(x, rows, j, low):
    if j == 4:
        return _grot(x, rows, 4)
    return jnp.where(low, _grot(x, rows, j), _grot(x, rows, -j))


def _blk_step(blk, rows, a0, pd, dird, ascm):
    j = 1 << pd
    pieces = []
    for b in range(0, rows, 2 * j):
        lo = blk[b:b + j, :]
        hi = blk[b + j:b + 2 * j, :]
        mn = jnp.minimum(lo, hi)
        mx = jnp.maximum(lo, hi)
        if dird is None:
            pieces.extend([mn, mx])
        elif dird >= 3:
            asc = (((a0 + b) >> dird) & 1) == 0
            pieces.extend([mn, mx] if asc else [mx, mn])
        else:
            am = jnp.concatenate([ascm[dird]] * (j // 8), axis=0)
            pieces.append(jnp.where(am, mn, mx))
            pieces.append(jnp.where(am, mx, mn))
    return jnp.concatenate(pieces, axis=0)


def _tc_body(lat_ref, out_ref, xs_ref):
    row = jax.lax.broadcasted_iota(jnp.int32, (_D, 1), 0)
    low_masks = [(row & (1 << pd)) == 0 for pd in range(3)]
    ascm = [((row[:8, :] >> d) & 1) == 0 for d in range(3)]

    x = lat_ref[...].T
    zl = jnp.sum(jnp.where(x == 0.0, 1.0, 0.0), axis=0, keepdims=True)
    l1 = jnp.sum(jnp.abs(x), axis=0, keepdims=True)

    sigma = ((row & 7) << 7) | (row >> 3)
    w = 2.0 * sigma.astype(jnp.float32) + float(1 - _D)

    for a in range(0, _D, _SB):
        blk = x[a:a + _SB, :]
        for s, p in ((0, 0), (1, 1), (1, 0), (2, 2), (2, 1), (2, 0),
                     (3, 3), (3, 2), (3, 1), (3, 0)):
            blk = _blk_step(blk, _SB, a, _phi(p), _phi(s + 1), ascm)
        xs_ref[pl.ds(a, _SB), :] = blk

    t_acc = jnp.zeros((1, _B), jnp.float32)

    def grot_chain(xv, rows, s, dird):
        for p in range(s, 6, -1):
            pd = _phi(p)
            partner = _xor_partner(xv, rows, 1 << pd, low_masks[pd][:rows, :])
            mn = jnp.minimum(xv, partner)
            mx = jnp.maximum(xv, partner)
            if dird is None:
                sel = low_masks[pd][:rows, :]
            else:
                sel = (low_masks[pd] == (((row >> dird) & 1) == 0))[:rows, :]
            xv = jnp.where(sel, mn, mx)
        return xv

    for s in range(4, 10):
        dird = _phi(s + 1) if s < 9 else None
        if s >= 7:
            lo = grot_chain(xs_ref[pl.ds(0, 512), :], 512, s, dird)
            hi = grot_chain(xs_ref[pl.ds(512, 512), :], 512, s, dird)
            mn = jnp.minimum(lo, hi)
            mx = jnp.maximum(lo, hi)
            if dird is None:
                xs_ref[pl.ds(0, 512), :] = mn
                xs_ref[pl.ds(512, 512), :] = mx
            else:
                am = (((row >> dird) & 1) == 0)[:512, :]
                xs_ref[pl.ds(0, 512), :] = jnp.where(am, mn, mx)
                xs_ref[pl.ds(512, 512), :] = jnp.where(am, mx, mn)
        for p in range(min(s, 6) if s < 7 else 5, 4, -1):
            j = 1 << _phi(p)
            for a in range(0, _D, 2 * j):
                lo = xs_ref[pl.ds(a, j), :]
                hi = xs_ref[pl.ds(a + j, j), :]
                mn = jnp.minimum(lo, hi)
                mx = jnp.maximum(lo, hi)
                if dird is None or dird >= 3:
                    asc = dird is None or ((a >> dird) & 1) == 0
                    xs_ref[pl.ds(a, j), :] = mn if asc else mx
                    xs_ref[pl.ds(a + j, j), :] = mx if asc else mn
                else:
                    amj = jnp.concatenate([ascm[dird]] * (j // 8), axis=0)
                    xs_ref[pl.ds(a, j), :] = jnp.where(amj, mn, mx)
                    xs_ref[pl.ds(a + j, j), :] = jnp.where(amj, mx, mn)
        for a in range(0, _D, 256):
            blk = xs_ref[pl.ds(a, 256), :]
            for p in (4, 3, 2, 1, 0):
                blk = _blk_step(blk, 256, a, _phi(p), dird, ascm)
            if s < 9:
                xs_ref[pl.ds(a, 256), :] = blk
            else:
                t_acc = t_acc + jnp.sum(blk * w[a:a + 256, :], axis=0,
                                        keepdims=True)

    m_vec = 2.0 * t_acc - 2.0 * zl * l1
    out_ref[...] = jnp.sum(m_vec, axis=1, keepdims=True) * (_COF1 / 2.0 / _B)


def kernel(latent):
    loss = pl.pallas_call(
        _tc_body,
        out_shape=jax.ShapeDtypeStruct((1, 1), jnp.float32),
        scratch_shapes=[pltpu.VMEM((_D, _B), jnp.float32)],
    )(latent)
    return loss.reshape(())
